# Initial kernel scaffold; baseline (speedup 1.0000x reference)
import jax
import jax.numpy as jnp
from jax.experimental import pallas as pl


def kernel(x, Wg, W1, W2):
    raise NotImplementedError("write your pallas kernel here")


# speedup vs baseline: 1.2922x; 1.2922x over previous
import functools

import jax
import jax.numpy as jnp
from jax.experimental import pallas as pl
from jax.experimental.pallas import tpu as pltpu

D_MODEL = 768
FF = 3072
N_EXPERT = 8
FF_BLK = 768
NF = FF // FF_BLK


def _dot_t(a, b):
    return jax.lax.dot_general(a, b, (((1,), (1,)), ((), ())),
                               preferred_element_type=jnp.float32)


def _moe_body(x_ref, wg_ref, w1_ref, w2_ref, out_ref, w_scr):
    e = pl.program_id(0)
    f = pl.program_id(1)

    @pl.when((e == 0) & (f == 0))
    def _init():
        scores = _dot_t(x_ref[...], wg_ref[...])
        m1 = jnp.max(scores, axis=-1, keepdims=True)
        col = jax.lax.broadcasted_iota(jnp.int32, scores.shape, 1)
        col1 = jnp.min(jnp.where(scores == m1, col, N_EXPERT), axis=-1,
                       keepdims=True)
        mask1 = col == col1
        neg = jnp.float32(-jnp.inf)
        scores2 = jnp.where(mask1, neg, scores)
        m2 = jnp.max(scores2, axis=-1, keepdims=True)
        col2 = jnp.min(jnp.where(scores2 == m2, col, N_EXPERT), axis=-1,
                       keepdims=True)
        mask2 = col == col2
        p1 = 1.0 / (1.0 + jnp.exp(m2 - m1))
        p2 = 1.0 - p1
        w_scr[...] = p1 * mask1.astype(jnp.float32) + p2 * mask2.astype(jnp.float32)
        out_ref[...] = jnp.zeros_like(out_ref)

    xw1 = _dot_t(x_ref[...], w1_ref[0])
    h = xw1 * jax.nn.sigmoid(xw1)
    part = _dot_t(h, w2_ref[0])
    onehot = (jax.lax.broadcasted_iota(jnp.int32, (N_EXPERT, 1), 0) == e
              ).astype(jnp.float32)
    w_e = jnp.dot(w_scr[...], onehot, preferred_element_type=jnp.float32)
    out_ref[...] += w_e * part


@functools.partial(jax.jit, static_argnames=())
def kernel(x, Wg, W1, W2):
    B, T, C = x.shape
    x_flat = x.reshape(T, C)
    out = pl.pallas_call(
        _moe_body,
        grid=(N_EXPERT, NF),
        in_specs=[
            pl.BlockSpec((T, C), lambda e, f: (0, 0)),
            pl.BlockSpec((N_EXPERT, C), lambda e, f: (0, 0)),
            pl.BlockSpec((1, FF_BLK, C), lambda e, f: (e, f, 0)),
            pl.BlockSpec((1, C, FF_BLK), lambda e, f: (e, 0, f)),
        ],
        out_specs=pl.BlockSpec((T, C), lambda e, f: (0, 0)),
        out_shape=jax.ShapeDtypeStruct((T, C), jnp.float32),
        scratch_shapes=[pltpu.VMEM((T, N_EXPERT), jnp.float32)],
    )(x_flat, Wg, W1, W2)
    return out.reshape(B, T, C)
